# baseline (speedup 1.0000x reference)
import functools

import jax
import jax.numpy as jnp
from jax import lax
from jax.experimental import pallas as pl
from jax.experimental.pallas import tpu as pltpu
from jax.experimental.pallas import tpu_sc as plsc

B = 16384
L = 50
D = 64
CLS = 10
V = 1000000
NC = 2
NS = 16
NW = NC * NS
ROWS_PER_W = B // NW
G = 2
IDX_PER_G = G * L
NGROUPS = ROWS_PER_W // G
NBUF = 4
DREG = D // 16

CW = 128
NFULL = V // CW
NTAIL = V - NFULL * CW
JFULL = NFULL // NW
NLEFT = NFULL - JFULL * NW
TBUF = 2


def _sc_transpose(table_t, tail_flat):
    mesh = plsc.VectorSubcoreMesh(core_axis_name="c", subcore_axis_name="s")

    @functools.partial(
        pl.kernel,
        out_type=jax.ShapeDtypeStruct((V * D,), jnp.float32),
        mesh=mesh,
        scratch_types=(
            [pltpu.VMEM((TBUF, D, CW + 1), jnp.float32),
             pltpu.VMEM((TBUF, CW * D), jnp.float32)]
            + [pltpu.SemaphoreType.DMA] * (2 * TBUF)
        ),
        compiler_params=pltpu.CompilerParams(
            use_tc_tiling_on_sc=True, needs_layout_passes=False),
    )
    def k(src_hbm, tail_hbm, out_hbm, ibuf, obuf, *sems):
        w = lax.axis_index("s") * NC + lax.axis_index("c")
        isems = sems[:TBUF]
        osems = sems[TBUF:]
        rowsel = lax.iota(jnp.int32, 16)

        def chunk_of(j):
            return (w + NW * j) * CW

        def fill(slot, v0):
            pltpu.async_copy(
                src_hbm.at[:, pl.ds(v0, CW)],
                ibuf.at[slot, :, pl.ds(0, CW)], isems[slot])

        UNR = 8
        rowsels = [rowsel + 16 * q for q in range(DREG)]

        def transpose_chunk(slot, v0):
            pltpu.make_async_copy(
                src_hbm.at[:, pl.ds(v0, CW)],
                ibuf.at[slot, :, pl.ds(0, CW)], isems[slot]).wait()

            def col(it, carry):
                k0 = it * UNR
                vals = []
                for c in range(UNR):
                    colsel = jnp.full((16,), k0 + c, dtype=jnp.int32)
                    for q in range(DREG):
                        vals.append(plsc.load_gather(
                            ibuf.at[slot], [rowsels[q], colsel]))
                base = k0 * D
                for c in range(UNR):
                    for q in range(DREG):
                        obuf[slot, pl.ds(base + c * D + 16 * q, 16)] = (
                            vals[c * DREG + q])
                return carry

            lax.fori_loop(0, CW // UNR, col, 0)
            pltpu.async_copy(
                obuf.at[slot], out_hbm.at[pl.ds(v0 * D, CW * D)],
                osems[slot])

        for s in range(TBUF):
            fill(s, chunk_of(s))

        def body(j0, carry):
            for s in range(TBUF):
                j = j0 * TBUF + s
                v0 = chunk_of(j)
                @pl.when(j >= TBUF)
                def _():
                    pltpu.make_async_copy(
                        obuf.at[s],
                        out_hbm.at[pl.ds((chunk_of(j - TBUF)) * D, CW * D)],
                        osems[s]).wait()

                transpose_chunk(s, v0)
                jn = j + TBUF

                @pl.when(jn < JFULL)
                def _():
                    fill(s, chunk_of(jn))
            return carry

        lax.fori_loop(0, JFULL // TBUF, body, 0)
        for s in range(TBUF):
            pltpu.make_async_copy(
                obuf.at[s],
                out_hbm.at[pl.ds(chunk_of(JFULL - TBUF + s) * D, CW * D)],
                osems[s]).wait()

        @pl.when(w < NLEFT)
        def _():
            v0 = (NFULL - NLEFT) * CW + w * CW
            fill(0, v0)
            transpose_chunk(0, v0)
            pltpu.make_async_copy(
                obuf.at[0], out_hbm.at[pl.ds(v0 * D, CW * D)],
                osems[0]).wait()

        @pl.when(w == NLEFT)
        def _():
            v0 = NFULL * CW
            pltpu.async_copy(
                tail_hbm, obuf.at[0, pl.ds(0, NTAIL * D)], isems[0])
            pltpu.make_async_copy(
                tail_hbm, obuf.at[0, pl.ds(0, NTAIL * D)], isems[0]).wait()
            pltpu.async_copy(
                obuf.at[0, pl.ds(0, NTAIL * D)],
                out_hbm.at[pl.ds(v0 * D, NTAIL * D)], osems[0])
            pltpu.make_async_copy(
                obuf.at[0, pl.ds(0, NTAIL * D)],
                out_hbm.at[pl.ds(v0 * D, NTAIL * D)], osems[0]).wait()

    return k(table_t, tail_flat)


def _sc_gather_sum(table, texts_r):
    mesh = plsc.VectorSubcoreMesh(core_axis_name="c", subcore_axis_name="s")

    @functools.partial(
        pl.kernel,
        out_type=jax.ShapeDtypeStruct((B, D), jnp.float32),
        mesh=mesh,
        scratch_types=(
            [pltpu.VMEM((NGROUPS, IDX_PER_G), jnp.int32),
             pltpu.VMEM((NBUF, IDX_PER_G, D), jnp.float32),
             pltpu.VMEM((ROWS_PER_W, D), jnp.float32)]
            + [pltpu.SemaphoreType.DMA] * NBUF
        ),
        compiler_params=pltpu.CompilerParams(use_tc_tiling_on_sc=False),
    )
    def k(table_hbm, texts_hbm, out_hbm, idx_v, rows_v, out_v, *sems):
        w = lax.axis_index("s") * NC + lax.axis_index("c")
        pltpu.sync_copy(texts_hbm.at[w], idx_v)
        for b in range(NBUF):
            pltpu.async_copy(table_hbm.at[idx_v.at[b]], rows_v.at[b], sems[b])

        def body(g0, carry):
            for b in range(NBUF):
                g = g0 * NBUF + b
                pltpu.make_async_copy(
                    table_hbm.at[idx_v.at[g]], rows_v.at[b], sems[b]).wait()
                for i in range(G):
                    accs = [rows_v[b, i * L, pl.ds(d * 16, 16)]
                            for d in range(DREG)]
                    for r in range(1, L):
                        for d in range(DREG):
                            accs[d] = accs[d] + rows_v[b, i * L + r,
                                                       pl.ds(d * 16, 16)]
                    row = G * g + i
                    for d in range(DREG):
                        out_v[row, pl.ds(d * 16, 16)] = accs[d]
                gn = g + NBUF

                @pl.when(gn < NGROUPS)
                def _():
                    pltpu.async_copy(
                        table_hbm.at[idx_v.at[gn]], rows_v.at[b], sems[b])
            return carry

        lax.fori_loop(0, NGROUPS // NBUF, body, 0)
        pltpu.sync_copy(out_v, out_hbm.at[pl.ds(w * ROWS_PER_W, ROWS_PER_W)])

    return k(table, texts_r)


def _tc_fc(x, wt, bias2d):
    tb = 2048

    def body(x_ref, w_ref, b_ref, o_ref):
        o_ref[...] = (
            jnp.dot(x_ref[...], w_ref[...], preferred_element_type=jnp.float32)
            * (1.0 / L)
            + b_ref[...]
        )

    return pl.pallas_call(
        body,
        grid=(B // tb,),
        in_specs=[
            pl.BlockSpec((tb, D), lambda i: (i, 0)),
            pl.BlockSpec((D, CLS), lambda i: (0, 0)),
            pl.BlockSpec((1, CLS), lambda i: (0, 0)),
        ],
        out_specs=pl.BlockSpec((tb, CLS), lambda i: (i, 0)),
        out_shape=jax.ShapeDtypeStruct((B, CLS), jnp.float32),
    )(x, wt, bias2d)


def kernel(texts, emb_table, fc_weight, fc_bias):
    idx = texts.astype(jnp.int32).reshape(NW, NGROUPS, IDX_PER_G)
    tail_flat = emb_table[NFULL * CW:].reshape(NTAIL * D)
    tbl_flat = _sc_transpose(emb_table.T, tail_flat)
    sums = _sc_gather_sum(tbl_flat.reshape(V, D), idx)
    return _tc_fc(sums, fc_weight.T, fc_bias.reshape(1, CLS))


# speedup vs baseline: 1.0193x; 1.0193x over previous
import functools

import jax
import jax.numpy as jnp
from jax import lax
from jax.experimental import pallas as pl
from jax.experimental.pallas import tpu as pltpu
from jax.experimental.pallas import tpu_sc as plsc

B = 16384
L = 50
D = 64
CLS = 10
V = 1000000
NC = 2
NS = 16
NW = NC * NS
ROWS_PER_W = B // NW
G = 2
IDX_PER_G = G * L
NGROUPS = ROWS_PER_W // G
NBUF = 4
DREG = D // 16

CW = 128
NFULL = V // CW
NTAIL = V - NFULL * CW
JFULL = NFULL // NW
NLEFT = NFULL - JFULL * NW
TBUF = 2


def _sc_transpose(table_t, tail_flat):
    mesh = plsc.VectorSubcoreMesh(core_axis_name="c", subcore_axis_name="s")

    @functools.partial(
        pl.kernel,
        out_type=jax.ShapeDtypeStruct((V * D,), jnp.float32),
        mesh=mesh,
        scratch_types=(
            [pltpu.VMEM((TBUF, D, CW + 1), jnp.float32),
             pltpu.VMEM((TBUF, CW * D), jnp.float32)]
            + [pltpu.SemaphoreType.DMA] * (2 * TBUF)
        ),
        compiler_params=pltpu.CompilerParams(
            use_tc_tiling_on_sc=True, needs_layout_passes=False),
    )
    def k(src_hbm, tail_hbm, out_hbm, ibuf, obuf, *sems):
        w = lax.axis_index("s") * NC + lax.axis_index("c")
        isems = sems[:TBUF]
        osems = sems[TBUF:]
        rowsel = lax.iota(jnp.int32, 16)

        def chunk_of(j):
            return (w + NW * j) * CW

        def fill(slot, v0):
            pltpu.async_copy(
                src_hbm.at[:, pl.ds(v0, CW)],
                ibuf.at[slot, :, pl.ds(0, CW)], isems[slot])

        UNR = 4
        rowsels = [rowsel + 16 * q for q in range(DREG)]

        def transpose_chunk(slot, v0):
            pltpu.make_async_copy(
                src_hbm.at[:, pl.ds(v0, CW)],
                ibuf.at[slot, :, pl.ds(0, CW)], isems[slot]).wait()

            def col(it, carry):
                k0 = it * UNR
                vals = []
                for c in range(UNR):
                    colsel = jnp.full((16,), k0 + c, dtype=jnp.int32)
                    for q in range(DREG):
                        vals.append(plsc.load_gather(
                            ibuf.at[slot], [rowsels[q], colsel]))
                base = k0 * D
                for c in range(UNR):
                    for q in range(DREG):
                        obuf[slot, pl.ds(base + c * D + 16 * q, 16)] = (
                            vals[c * DREG + q])
                return carry

            lax.fori_loop(0, CW // UNR, col, 0)
            pltpu.async_copy(
                obuf.at[slot], out_hbm.at[pl.ds(v0 * D, CW * D)],
                osems[slot])

        for s in range(TBUF):
            fill(s, chunk_of(s))

        def body(j0, carry):
            for s in range(TBUF):
                j = j0 * TBUF + s
                v0 = chunk_of(j)
                @pl.when(j >= TBUF)
                def _():
                    pltpu.make_async_copy(
                        obuf.at[s],
                        out_hbm.at[pl.ds((chunk_of(j - TBUF)) * D, CW * D)],
                        osems[s]).wait()

                transpose_chunk(s, v0)
                jn = j + TBUF

                @pl.when(jn < JFULL)
                def _():
                    fill(s, chunk_of(jn))
            return carry

        lax.fori_loop(0, JFULL // TBUF, body, 0)
        for s in range(TBUF):
            pltpu.make_async_copy(
                obuf.at[s],
                out_hbm.at[pl.ds(chunk_of(JFULL - TBUF + s) * D, CW * D)],
                osems[s]).wait()

        @pl.when(w < NLEFT)
        def _():
            v0 = (NFULL - NLEFT) * CW + w * CW
            fill(0, v0)
            transpose_chunk(0, v0)
            pltpu.make_async_copy(
                obuf.at[0], out_hbm.at[pl.ds(v0 * D, CW * D)],
                osems[0]).wait()

        @pl.when(w == NLEFT)
        def _():
            v0 = NFULL * CW
            pltpu.async_copy(
                tail_hbm, obuf.at[0, pl.ds(0, NTAIL * D)], isems[0])
            pltpu.make_async_copy(
                tail_hbm, obuf.at[0, pl.ds(0, NTAIL * D)], isems[0]).wait()
            pltpu.async_copy(
                obuf.at[0, pl.ds(0, NTAIL * D)],
                out_hbm.at[pl.ds(v0 * D, NTAIL * D)], osems[0])
            pltpu.make_async_copy(
                obuf.at[0, pl.ds(0, NTAIL * D)],
                out_hbm.at[pl.ds(v0 * D, NTAIL * D)], osems[0]).wait()

    return k(table_t, tail_flat)


def _sc_gather_sum(table, texts_r):
    mesh = plsc.VectorSubcoreMesh(core_axis_name="c", subcore_axis_name="s")

    @functools.partial(
        pl.kernel,
        out_type=jax.ShapeDtypeStruct((B, D), jnp.float32),
        mesh=mesh,
        scratch_types=(
            [pltpu.VMEM((NGROUPS, IDX_PER_G), jnp.int32),
             pltpu.VMEM((NBUF, IDX_PER_G, D), jnp.float32),
             pltpu.VMEM((ROWS_PER_W, D), jnp.float32)]
            + [pltpu.SemaphoreType.DMA] * NBUF
        ),
        compiler_params=pltpu.CompilerParams(use_tc_tiling_on_sc=False),
    )
    def k(table_hbm, texts_hbm, out_hbm, idx_v, rows_v, out_v, *sems):
        w = lax.axis_index("s") * NC + lax.axis_index("c")
        pltpu.sync_copy(texts_hbm.at[w], idx_v)
        for b in range(NBUF):
            pltpu.async_copy(table_hbm.at[idx_v.at[b]], rows_v.at[b], sems[b])

        def body(g0, carry):
            for b in range(NBUF):
                g = g0 * NBUF + b
                pltpu.make_async_copy(
                    table_hbm.at[idx_v.at[g]], rows_v.at[b], sems[b]).wait()
                for i in range(G):
                    accs = [rows_v[b, i * L, pl.ds(d * 16, 16)]
                            for d in range(DREG)]
                    for r in range(1, L):
                        for d in range(DREG):
                            accs[d] = accs[d] + rows_v[b, i * L + r,
                                                       pl.ds(d * 16, 16)]
                    row = G * g + i
                    for d in range(DREG):
                        out_v[row, pl.ds(d * 16, 16)] = accs[d]
                gn = g + NBUF

                @pl.when(gn < NGROUPS)
                def _():
                    pltpu.async_copy(
                        table_hbm.at[idx_v.at[gn]], rows_v.at[b], sems[b])
            return carry

        lax.fori_loop(0, NGROUPS // NBUF, body, 0)
        pltpu.sync_copy(out_v, out_hbm.at[pl.ds(w * ROWS_PER_W, ROWS_PER_W)])

    return k(table, texts_r)


def _tc_fc(x, wt, bias2d):
    tb = 2048

    def body(x_ref, w_ref, b_ref, o_ref):
        o_ref[...] = (
            jnp.dot(x_ref[...], w_ref[...], preferred_element_type=jnp.float32)
            * (1.0 / L)
            + b_ref[...]
        )

    return pl.pallas_call(
        body,
        grid=(B // tb,),
        in_specs=[
            pl.BlockSpec((tb, D), lambda i: (i, 0)),
            pl.BlockSpec((D, CLS), lambda i: (0, 0)),
            pl.BlockSpec((1, CLS), lambda i: (0, 0)),
        ],
        out_specs=pl.BlockSpec((tb, CLS), lambda i: (i, 0)),
        out_shape=jax.ShapeDtypeStruct((B, CLS), jnp.float32),
    )(x, wt, bias2d)


def kernel(texts, emb_table, fc_weight, fc_bias):
    idx = texts.astype(jnp.int32).reshape(NW, NGROUPS, IDX_PER_G)
    tail_flat = emb_table[NFULL * CW:].reshape(NTAIL * D)
    tbl_flat = _sc_transpose(emb_table.T, tail_flat)
    sums = _sc_gather_sum(tbl_flat.reshape(V, D), idx)
    return _tc_fc(sums, fc_weight.T, fc_bias.reshape(1, CLS))


# speedup vs baseline: 1.6312x; 1.6004x over previous
import functools

import jax
import jax.numpy as jnp
from jax import lax
from jax.experimental import pallas as pl
from jax.experimental.pallas import tpu as pltpu
from jax.experimental.pallas import tpu_sc as plsc

B = 16384
L = 50
D = 64
CLS = 10
NC = 2
NS = 16
NW = NC * NS
ROWS_PER_W = B // NW
G = 2
IDX_PER_G = G * L
NGROUPS = ROWS_PER_W // G
NBUF = 8
DREG = D // 16


def _sc_gather_sum(table, texts_r):
    mesh = plsc.VectorSubcoreMesh(core_axis_name="c", subcore_axis_name="s")

    @functools.partial(
        pl.kernel,
        out_type=jax.ShapeDtypeStruct((B, D), jnp.float32),
        mesh=mesh,
        scratch_types=(
            [pltpu.VMEM((NGROUPS, IDX_PER_G), jnp.int32),
             pltpu.VMEM((NBUF, IDX_PER_G, D), jnp.float32),
             pltpu.VMEM((ROWS_PER_W, D), jnp.float32)]
            + [pltpu.SemaphoreType.DMA] * NBUF
        ),
        compiler_params=pltpu.CompilerParams(use_tc_tiling_on_sc=False),
    )
    def k(table_hbm, texts_hbm, out_hbm, idx_v, rows_v, out_v, *sems):
        w = lax.axis_index("s") * NC + lax.axis_index("c")
        pltpu.sync_copy(texts_hbm.at[w], idx_v)
        for b in range(NBUF):
            pltpu.async_copy(table_hbm.at[idx_v.at[b]], rows_v.at[b], sems[b])

        def body(g0, carry):
            for b in range(NBUF):
                g = g0 * NBUF + b
                pltpu.make_async_copy(
                    table_hbm.at[idx_v.at[g]], rows_v.at[b], sems[b]).wait()
                for i in range(G):
                    accs = [rows_v[b, i * L, pl.ds(d * 16, 16)]
                            for d in range(DREG)]
                    for r in range(1, L):
                        for d in range(DREG):
                            accs[d] = accs[d] + rows_v[b, i * L + r,
                                                       pl.ds(d * 16, 16)]
                    row = G * g + i
                    for d in range(DREG):
                        out_v[row, pl.ds(d * 16, 16)] = accs[d]
                gn = g + NBUF

                @pl.when(gn < NGROUPS)
                def _():
                    pltpu.async_copy(
                        table_hbm.at[idx_v.at[gn]], rows_v.at[b], sems[b])
            return carry

        lax.fori_loop(0, NGROUPS // NBUF, body, 0)
        pltpu.sync_copy(out_v, out_hbm.at[pl.ds(w * ROWS_PER_W, ROWS_PER_W)])

    return k(table, texts_r)


def _tc_fc(x, wt, bias2d):
    tb = 2048

    def body(x_ref, w_ref, b_ref, o_ref):
        o_ref[...] = (
            jnp.dot(x_ref[...], w_ref[...], preferred_element_type=jnp.float32)
            * (1.0 / L)
            + b_ref[...]
        )

    return pl.pallas_call(
        body,
        grid=(B // tb,),
        in_specs=[
            pl.BlockSpec((tb, D), lambda i: (i, 0)),
            pl.BlockSpec((D, CLS), lambda i: (0, 0)),
            pl.BlockSpec((1, CLS), lambda i: (0, 0)),
        ],
        out_specs=pl.BlockSpec((tb, CLS), lambda i: (i, 0)),
        out_shape=jax.ShapeDtypeStruct((B, CLS), jnp.float32),
    )(x, wt, bias2d)


def kernel(texts, emb_table, fc_weight, fc_bias):
    idx = texts.astype(jnp.int32).reshape(NW, NGROUPS, IDX_PER_G)
    sums = _sc_gather_sum(emb_table, idx)
    return _tc_fc(sums, fc_weight.T, fc_bias.reshape(1, CLS))


# speedup vs baseline: 1.7114x; 1.0491x over previous
import functools

import jax
import jax.numpy as jnp
from jax import lax
from jax.experimental import pallas as pl
from jax.experimental.pallas import tpu as pltpu
from jax.experimental.pallas import tpu_sc as plsc

B = 16384
L = 50
D = 64
CLS = 10
NC = 2
NS = 16
NW = NC * NS
ROWS_PER_W = B // NW
G = 2
IDX_PER_G = G * L
NGROUPS = ROWS_PER_W // G
NBUF = 4
DREG = D // 16


def _sc_gather_sum(table, texts_r):
    mesh = plsc.VectorSubcoreMesh(core_axis_name="c", subcore_axis_name="s")

    @functools.partial(
        pl.kernel,
        out_type=jax.ShapeDtypeStruct((B, D), jnp.float32),
        mesh=mesh,
        scratch_types=(
            [pltpu.VMEM((NGROUPS, IDX_PER_G), jnp.int32),
             pltpu.VMEM((NBUF, IDX_PER_G, D), jnp.float32),
             pltpu.VMEM((ROWS_PER_W, D), jnp.float32)]
            + [pltpu.SemaphoreType.DMA] * NBUF
        ),
        compiler_params=pltpu.CompilerParams(use_tc_tiling_on_sc=False),
    )
    def k(table_hbm, texts_hbm, out_hbm, idx_v, rows_v, out_v, *sems):
        w = lax.axis_index("s") * NC + lax.axis_index("c")
        pltpu.sync_copy(texts_hbm.at[w], idx_v)
        for b in range(NBUF):
            pltpu.async_copy(table_hbm.at[idx_v.at[b]], rows_v.at[b], sems[b])

        def body(g0, carry):
            for b in range(NBUF):
                g = g0 * NBUF + b
                pltpu.make_async_copy(
                    table_hbm.at[idx_v.at[g]], rows_v.at[b], sems[b]).wait()
                for i in range(G):
                    accs = [rows_v[b, i * L, pl.ds(d * 16, 16)]
                            for d in range(DREG)]
                    for r in range(1, L):
                        for d in range(DREG):
                            accs[d] = accs[d] + rows_v[b, i * L + r,
                                                       pl.ds(d * 16, 16)]
                    row = G * g + i
                    for d in range(DREG):
                        out_v[row, pl.ds(d * 16, 16)] = accs[d]
                gn = g + NBUF

                @pl.when(gn < NGROUPS)
                def _():
                    pltpu.async_copy(
                        table_hbm.at[idx_v.at[gn]], rows_v.at[b], sems[b])
            return carry

        lax.fori_loop(0, NGROUPS // NBUF, body, 0)
        pltpu.sync_copy(out_v, out_hbm.at[pl.ds(w * ROWS_PER_W, ROWS_PER_W)])

    return k(table, texts_r)


def _tc_fc(x, wt, bias2d):
    tb = 2048

    def body(x_ref, w_ref, b_ref, o_ref):
        o_ref[...] = (
            jnp.dot(x_ref[...], w_ref[...], preferred_element_type=jnp.float32)
            * (1.0 / L)
            + b_ref[...]
        )

    return pl.pallas_call(
        body,
        grid=(B // tb,),
        in_specs=[
            pl.BlockSpec((tb, D), lambda i: (i, 0)),
            pl.BlockSpec((D, CLS), lambda i: (0, 0)),
            pl.BlockSpec((1, CLS), lambda i: (0, 0)),
        ],
        out_specs=pl.BlockSpec((tb, CLS), lambda i: (i, 0)),
        out_shape=jax.ShapeDtypeStruct((B, CLS), jnp.float32),
    )(x, wt, bias2d)


def kernel(texts, emb_table, fc_weight, fc_bias):
    idx = texts.astype(jnp.int32).reshape(NW, NGROUPS, IDX_PER_G)
    sums = _sc_gather_sum(emb_table, idx)
    return _tc_fc(sums, fc_weight.T, fc_bias.reshape(1, CLS))
